# baseline (speedup 1.0000x reference)
import functools

import jax
import jax.numpy as jnp
from jax import lax
from jax.experimental import pallas as pl
from jax.experimental.pallas import tpu as pltpu
from jax.experimental.pallas import tpu_sc as plsc

DP = 128


def kernel(x, table):
    B, S = x.shape
    V, D = table.shape
    N = B * S

    info = plsc.get_sparse_core_info()
    NC, NS = info.num_cores, info.num_subcores
    NW = NC * NS
    CHUNK = 128
    per_w = N // NW
    n_chunks = per_w // CHUNK

    table_p = jnp.pad(table, ((0, 0), (0, DP - D)))
    idx = x.reshape(NW, n_chunks, CHUNK)
    mesh = plsc.VectorSubcoreMesh(core_axis_name="c", subcore_axis_name="s")

    @functools.partial(
        pl.kernel,
        mesh=mesh,
        out_type=jax.ShapeDtypeStruct((NW, per_w, DP), jnp.float32),
        scratch_types=[
            pltpu.VMEM((n_chunks, CHUNK), jnp.int32),
            pltpu.VMEM((CHUNK, DP), jnp.float32),
            pltpu.SemaphoreType.DMA,
        ],
    )
    def emb(idx_hbm, table_hbm, out_hbm, idx_v, rows_v, sem):
        wid = lax.axis_index("s") * NC + lax.axis_index("c")
        pltpu.sync_copy(idx_hbm.at[wid], idx_v)

        def body(c, carry):
            pltpu.async_copy(table_hbm.at[idx_v.at[c]], rows_v, sem).wait()
            pltpu.sync_copy(rows_v, out_hbm.at[wid, pl.ds(c * CHUNK, CHUNK)])
            return carry

        lax.fori_loop(0, n_chunks, body, 0)

    out = emb(idx, table_p)
    return out.reshape(N, DP)[:, :D].reshape(B, S, D)
